# baseline (speedup 1.0000x reference)
import functools

import jax
import jax.numpy as jnp
from jax.experimental import pallas as pl
from jax.experimental.pallas import tpu as pltpu

N = 10000
E = 160000
B = 8
L = 128
D_IN = 12
HID = 512
EMB = 128
LAT = 256
N_LAYERS = 8
HEADS = 8
HEAD_DIM = 64
DEC_LAYERS = 3
N_TOKENS = 512
MAX_BLOCKS = 2048

ROW_BLK = 1000


def _mm_body(h_ref, w_ref, b_ref, o_ref):
    o_ref[...] = (
        jnp.dot(h_ref[...], w_ref[...], preferred_element_type=jnp.float32)
        + b_ref[...]
    )


def _matmul_bias(h, w, b):
    n, k = h.shape
    m = w.shape[1]
    grid = (n // ROW_BLK,)
    return pl.pallas_call(
        _mm_body,
        grid=grid,
        in_specs=[
            pl.BlockSpec((ROW_BLK, k), lambda i: (i, 0)),
            pl.BlockSpec((k, m), lambda i: (0, 0)),
            pl.BlockSpec((1, m), lambda i: (0, 0)),
        ],
        out_specs=pl.BlockSpec((ROW_BLK, m), lambda i: (i, 0)),
        out_shape=jax.ShapeDtypeStruct((n, m), jnp.float32),
    )(h, w, b.reshape(1, m))


def _ln(x, g, b):
    m = jnp.mean(x, -1, keepdims=True)
    v = jnp.mean((x - m) ** 2, -1, keepdims=True)
    return g * (x - m) / jnp.sqrt(v + 1e-5) + b


def _gelu(x):
    return jax.nn.gelu(x, approximate=False)


def kernel(x, node_type, block_data, port_direction, edge_type, edge_index,
           batch, decoder_input_tokens, eps, params):
    p = params
    nt = jnp.remainder(jnp.maximum(node_type, 0), 2)
    bd = jnp.remainder(jnp.maximum(block_data, 0), 64)
    pd = jnp.remainder(jnp.maximum(port_direction + 1, 0), 8)
    et = jnp.remainder(jnp.maximum(edge_type, 0), 2)
    feats = jnp.concatenate(
        [x.astype(jnp.float32), p['node_type_emb'][nt], p['block_data_emb'][bd],
         p['port_dir_emb'][pd]], axis=-1)
    h = feats @ p['Wp'] + p['bp']
    e_base = p['edge_type_emb'][et]
    src = edge_index[0]
    dst = edge_index[1]

    for l in range(N_LAYERS):
        residual = h
        w_all = jnp.concatenate(
            [p['Wq'][l], p['Wk'][l], p['Wv'][l], p['Wskip'][l]], axis=1)
        b_all = jnp.concatenate(
            [p['bq'][l], p['bk'][l], p['bv'][l], p['bskip'][l]], axis=0)
        qkvs = _matmul_bias(h, w_all, b_all)
        qn = qkvs[:, 0:HID]
        kn = qkvs[:, HID:2 * HID]
        vn = qkvs[:, 2 * HID:3 * HID]
        x_r = qkvs[:, 3 * HID:4 * HID]

        e = e_base @ p['We'][l] + p['be'][l]
        q = qn[dst].reshape(E, HEADS, HEAD_DIM)
        k = (kn[src] + e).reshape(E, HEADS, HEAD_DIM)
        v = (vn[src] + e).reshape(E, HEADS, HEAD_DIM)
        alpha = jnp.sum(q * k, axis=-1) / 8.0
        m = jax.ops.segment_max(alpha, dst, num_segments=N)
        m = jnp.where(jnp.isfinite(m), m, 0.0)
        ex = jnp.exp(alpha - m[dst])
        s = jax.ops.segment_sum(ex, dst, num_segments=N)
        alpha = ex / (s[dst] + 1e-16)
        out = jax.ops.segment_sum((v * alpha[:, :, None]).reshape(E, HID), dst,
                                  num_segments=N)
        beta = jax.nn.sigmoid(
            jnp.concatenate([out, x_r, out - x_r], axis=-1) @ p['Wbeta'][l])
        out = beta * x_r + (1.0 - beta) * out
        h = _gelu(_ln(out + residual, p['ln_g'][l], p['ln_b'][l]))

    cnt = jnp.maximum(
        jax.ops.segment_sum(jnp.ones((N,), jnp.float32), batch, num_segments=B),
        1.0)[:, None]
    mean_pool = jax.ops.segment_sum(h, batch, num_segments=B) / cnt
    max_pool = jax.ops.segment_max(h, batch, num_segments=B)
    pooled = jnp.concatenate([mean_pool, max_pool], axis=-1)
    mu = pooled @ p['Wmu'] + p['bmu']
    logvar = pooled @ p['Wlv'] + p['blv']
    z = mu + eps * jnp.exp(0.5 * logvar)
    tok = p['tok_emb'][decoder_input_tokens]
    init = (z @ p['Wl2d'] + p['bl2d']).reshape(B, DEC_LAYERS, HID).transpose(1, 0, 2)
    xs = tok.transpose(1, 0, 2)
    for l in range(DEC_LAYERS):
        def step(hprev, x_t, Wih=p['Wih'][l], Whh=p['Whh'][l], bih=p['bih'][l],
                 bhh=p['bhh'][l]):
            gi = x_t @ Wih.T + bih
            gh = hprev @ Whh.T + bhh
            i_r, i_z, i_n = jnp.split(gi, 3, axis=-1)
            h_r, h_z, h_n = jnp.split(gh, 3, axis=-1)
            r = jax.nn.sigmoid(i_r + h_r)
            zg = jax.nn.sigmoid(i_z + h_z)
            n = jnp.tanh(i_n + r * h_n)
            hn = (1.0 - zg) * n + zg * hprev
            return hn, hn
        _, xs = jax.lax.scan(step, init[l], xs)
    out_seq = _ln(xs.transpose(1, 0, 2), p['dn_g'], p['dn_b'])
    token_logits = out_seq @ p['Wtok'] + p['btok']
    size_logits = _gelu(z @ p['Ws1'] + p['bs1']) @ p['Ws2'] + p['bs2']
    return (token_logits, size_logits, mu, logvar, z)


# speedup vs baseline: 1.5694x; 1.5694x over previous
import functools

import jax
import jax.numpy as jnp
import numpy as _np
from jax import lax
from jax.experimental import pallas as pl
from jax.experimental.pallas import tpu as pltpu
from jax.experimental.pallas import tpu_sc as plsc

N = 10000
E = 160000
B = 8
L = 128
D_IN = 12
HID = 512
EMB = 128
LAT = 256
N_LAYERS = 8
HEADS = 8
HEAD_DIM = 64
DEC_LAYERS = 3
N_TOKENS = 512
MAX_BLOCKS = 2048

ROW_BLK = 1000

PERM = _np.zeros(512, _np.int32)
for _w in range(32):
    for _l in range(16):
        _h = _l if _l < 8 else 15 - _l
        _d = 2 * _w if _l < 8 else 2 * _w + 1
        PERM[16 * _w + _l] = 64 * _h + _d

NC = 2
NS = 16
NW = NC * NS
BA = 32
RNG = 312
NTAIL = N - NW * RNG
SLAB = RNG + NTAIL


def _proj_body(h_ref, w_ref, b_ref, ep_ref, ev_ref, q_ref, k2_ref, va_ref,
               vb_ref, xr_ref):
    big = (jnp.dot(h_ref[...], w_ref[...], preferred_element_type=jnp.float32)
           + b_ref[...])
    q_ref[...] = big[:, 0:HID]
    k2_ref[...] = big[None, :, HID:2 * HID] + ep_ref[...][:, None, :]
    va_ref[...] = (big[None, :, 2 * HID:2 * HID + 256]
                   + ev_ref[...][:, None, 0:256])
    vb_ref[...] = (big[None, :, 2 * HID + 256:3 * HID]
                   + ev_ref[...][:, None, 256:512])
    xr_ref[...] = big[:, 3 * HID:4 * HID]


def _fused_proj(h, w, b, etabP, etabV):
    n, kdim = h.shape
    mdim = w.shape[1]
    grid = (n // ROW_BLK,)
    f32 = jnp.float32
    return pl.pallas_call(
        _proj_body,
        grid=grid,
        in_specs=[
            pl.BlockSpec((ROW_BLK, kdim), lambda i: (i, 0)),
            pl.BlockSpec((kdim, mdim), lambda i: (0, 0)),
            pl.BlockSpec((1, mdim), lambda i: (0, 0)),
            pl.BlockSpec((2, HID), lambda i: (0, 0)),
            pl.BlockSpec((2, HID), lambda i: (0, 0)),
        ],
        out_specs=[
            pl.BlockSpec((ROW_BLK, HID), lambda i: (i, 0)),
            pl.BlockSpec((2, ROW_BLK, HID), lambda i: (0, i, 0)),
            pl.BlockSpec((2, ROW_BLK, 256), lambda i: (0, i, 0)),
            pl.BlockSpec((2, ROW_BLK, 256), lambda i: (0, i, 0)),
            pl.BlockSpec((ROW_BLK, HID), lambda i: (i, 0)),
        ],
        out_shape=[
            jax.ShapeDtypeStruct((n, HID), f32),
            jax.ShapeDtypeStruct((2, n, HID), f32),
            jax.ShapeDtypeStruct((2, n, 256), f32),
            jax.ShapeDtypeStruct((2, n, 256), f32),
            jax.ShapeDtypeStruct((n, HID), f32),
        ],
    )(h, w, b.reshape(1, mdim), etabP, etabV)


def _worker_bounds(bndv, wid):
    lo = jnp.int32(0)
    hi = jnp.int32(0)
    for part in range(4):
        vec = bndv[pl.ds(16 * part, 16)]
        for i in range(16):
            idx = 16 * part + i
            if idx < 32:
                lo = jnp.where(wid == idx, vec[i], lo)
            else:
                hi = jnp.where(wid == (idx - 32), vec[i], hi)
    return lo, hi


def _edge_alpha_body(q_h, k2_h, src2_h, dst_h, bnd_h, ex_h, s_o,
                     srcv, dstv, bndv, qb, kb, exout, s_slab, sem):
    cid = lax.axis_index("c")
    sid = lax.axis_index("s")
    wid = sid * NC + cid
    node_lo = wid * RNG

    lane = lax.broadcasted_iota(jnp.int32, (16,), 0)
    lm8f = jnp.where(lane < 8, 1.0, 0.0).astype(jnp.float32)
    zeros16 = jnp.zeros((16,), jnp.float32)

    pltpu.sync_copy(bnd_h, bndv)
    lo_e, hi_e = _worker_bounds(bndv, wid)

    def zrow(r, c2):
        s_slab[r, :] = zeros16
        return c2

    lax.fori_loop(0, SLAB, zrow, 0)

    b_lo = lax.div(lo_e, BA)
    b_hi = lax.div(hi_e - 1, BA) + 1
    b_hi = jnp.maximum(b_hi, b_lo)

    def batch(bi, carry):
        base = bi * BA
        pltpu.sync_copy(dst_h.at[pl.ds(base, BA)], dstv)
        pltpu.sync_copy(src2_h.at[pl.ds(base, BA)], srcv)
        pltpu.async_copy(q_h.at[dstv], qb, sem).wait()
        pltpu.async_copy(k2_h.at[srcv], kb, sem).wait()
        for g in range(BA // 16):
            dx = dstv[pl.ds(16 * g, 16)]
            for i2 in range(16):
                j = 16 * g + i2
                acc = qb[j, pl.ds(0, 16)] * kb[j, pl.ds(0, 16)]
                for w in range(1, 32):
                    acc = acc + (qb[j, pl.ds(16 * w, 16)]
                                 * kb[j, pl.ds(16 * w, 16)])
                raw = acc + jnp.flip(acc)
                exr = jnp.exp(raw * 0.125) * lm8f
                exout[j, :] = exr
                eidx = base + j
                valid = (eidx >= lo_e) & (eidx < hi_e)
                row = dx[i2] - node_lo

                @pl.when(valid)
                def _(row=row, exr=exr):
                    plsc.addupdate(s_slab.at[row], exr)
        pltpu.sync_copy(exout, ex_h.at[pl.ds(base, BA)])
        return carry

    lax.fori_loop(b_lo, b_hi, batch, 0)

    pltpu.sync_copy(s_slab.at[pl.ds(0, RNG)], s_o.at[pl.ds(node_lo, RNG)])

    @pl.when(wid == NW - 1)
    def _():
        pltpu.sync_copy(s_slab.at[pl.ds(RNG, NTAIL)],
                        s_o.at[pl.ds(NW * RNG, NTAIL)])


def _edge_spread_body(va_h, vb_h, src2_h, dst_h, bnd_h, ex_h, acc_o,
                      srcv, dstv, bndv, vbuf, exv, acc_slab, sem):
    cid = lax.axis_index("c")
    sid = lax.axis_index("s")
    wid = sid * NC + cid
    node_lo = wid * RNG

    zeros16 = jnp.zeros((16,), jnp.float32)

    pltpu.sync_copy(bnd_h, bndv)
    lo_e, hi_e = _worker_bounds(bndv, wid)
    b_lo = lax.div(lo_e, BA)
    b_hi = lax.div(hi_e - 1, BA) + 1
    b_hi = jnp.maximum(b_hi, b_lo)

    for p, v_h in enumerate((va_h, vb_h)):

        def zrow(r, c2):
            for t in range(16):
                acc_slab[r, pl.ds(16 * t, 16)] = zeros16
            return c2

        lax.fori_loop(0, SLAB, zrow, 0)

        def batch(bi, carry, p=p, v_h=v_h):
            base = bi * BA
            pltpu.sync_copy(dst_h.at[pl.ds(base, BA)], dstv)
            pltpu.sync_copy(src2_h.at[pl.ds(base, BA)], srcv)
            pltpu.sync_copy(ex_h.at[pl.ds(base, BA)], exv)
            pltpu.async_copy(v_h.at[srcv], vbuf, sem).wait()
            for g in range(BA // 16):
                dx = dstv[pl.ds(16 * g, 16)]
                for i2 in range(16):
                    j = 16 * g + i2
                    exrow = exv[j, :]
                    eidx = base + j
                    valid = (eidx >= lo_e) & (eidx < hi_e)
                    row = dx[i2] - node_lo

                    @pl.when(valid)
                    def _(row=row, exrow=exrow, j=j, p=p):
                        for t in range(16):
                            scale = exrow[4 * p + t // 4]
                            plsc.addupdate(
                                acc_slab.at[row, pl.ds(16 * t, 16)],
                                vbuf[j, pl.ds(16 * t, 16)] * scale)
            return carry

        lax.fori_loop(b_lo, b_hi, batch, 0)

        pltpu.sync_copy(acc_slab.at[pl.ds(0, RNG)],
                        acc_o.at[p, pl.ds(node_lo, RNG)])

        @pl.when(wid == NW - 1)
        def _(p=p):
            pltpu.sync_copy(acc_slab.at[pl.ds(RNG, NTAIL)],
                            acc_o.at[p, pl.ds(NW * RNG, NTAIL)])


def _edge_aggregate(qp, k2, va, vb, src2, dsts, bnd):
    mesh = plsc.VectorSubcoreMesh(core_axis_name="c", subcore_axis_name="s",
                                  num_cores=NC, num_subcores=NS)
    f32 = jnp.float32
    run_a = pl.kernel(
        _edge_alpha_body,
        out_type=(
            jax.ShapeDtypeStruct((E, 16), f32),
            jax.ShapeDtypeStruct((N, 16), f32),
        ),
        mesh=mesh,
        scratch_types=[
            pltpu.VMEM((BA,), jnp.int32),
            pltpu.VMEM((BA,), jnp.int32),
            pltpu.VMEM((64,), jnp.int32),
            pltpu.VMEM((BA, HID), f32),
            pltpu.VMEM((BA, HID), f32),
            pltpu.VMEM((BA, 16), f32),
            pltpu.VMEM((SLAB, 16), f32),
            pltpu.SemaphoreType.DMA,
        ],
    )
    ex, s = run_a(qp, k2, src2, dsts, bnd)
    run_b = pl.kernel(
        _edge_spread_body,
        out_type=jax.ShapeDtypeStruct((2, N, 256), f32),
        mesh=mesh,
        scratch_types=[
            pltpu.VMEM((BA,), jnp.int32),
            pltpu.VMEM((BA,), jnp.int32),
            pltpu.VMEM((64,), jnp.int32),
            pltpu.VMEM((BA, 256), f32),
            pltpu.VMEM((BA, 16), f32),
            pltpu.VMEM((SLAB, 256), f32),
            pltpu.SemaphoreType.DMA,
        ],
    )
    acc = run_b(va, vb, src2, dsts, bnd, ex)
    return ex, s, acc


def _ln(x, g, b):
    m = jnp.mean(x, -1, keepdims=True)
    v = jnp.mean((x - m) ** 2, -1, keepdims=True)
    return g * (x - m) / jnp.sqrt(v + 1e-5) + b


def _gelu(x):
    return jax.nn.gelu(x, approximate=False)


def kernel(x, node_type, block_data, port_direction, edge_type, edge_index,
           batch, decoder_input_tokens, eps, params):
    p = params
    nt = jnp.remainder(jnp.maximum(node_type, 0), 2)
    bd = jnp.remainder(jnp.maximum(block_data, 0), 64)
    pd = jnp.remainder(jnp.maximum(port_direction + 1, 0), 8)
    et = jnp.remainder(jnp.maximum(edge_type, 0), 2).astype(jnp.int32)
    feats = jnp.concatenate(
        [x.astype(jnp.float32), p['node_type_emb'][nt], p['block_data_emb'][bd],
         p['port_dir_emb'][pd]], axis=-1)
    h = feats @ p['Wp'] + p['bp']
    src = edge_index[0].astype(jnp.int32)
    dst = edge_index[1].astype(jnp.int32)

    order = jnp.argsort(dst)
    dsts = dst[order]
    srcs = src[order]
    ets = et[order]
    src2 = srcs + ets * N
    cuts = jnp.searchsorted(dsts, (jnp.arange(1, NW) * RNG).astype(jnp.int32))
    bnd = jnp.concatenate([
        jnp.zeros((1,), jnp.int32), cuts.astype(jnp.int32),
        cuts.astype(jnp.int32), jnp.full((1,), E, jnp.int32)]).astype(jnp.int32)

    perm = jnp.asarray(PERM)

    for l in range(N_LAYERS):
        residual = h
        w_all = jnp.concatenate(
            [p['Wq'][l][:, perm], p['Wk'][l][:, perm], p['Wv'][l],
             p['Wskip'][l]], axis=1)
        b_all = jnp.concatenate(
            [p['bq'][l][perm], p['bk'][l][perm], p['bv'][l], p['bskip'][l]],
            axis=0)
        etab = p['edge_type_emb'] @ p['We'][l] + p['be'][l]
        etabP = etab[:, perm]

        qp, k2, va, vb, x_r = _fused_proj(h, w_all, b_all, etabP, etab)
        ex, s, acc = _edge_aggregate(
            qp, k2.reshape(2 * N, HID), va.reshape(2 * N, 256),
            vb.reshape(2 * N, 256), src2, dsts, bnd)

        s8 = s[:, 0:8]
        num = jnp.concatenate([acc[0], acc[1]], axis=-1)
        out = num / (jnp.repeat(s8, HEAD_DIM, axis=1) + 1e-16)

        beta = jax.nn.sigmoid(
            jnp.concatenate([out, x_r, out - x_r], axis=-1) @ p['Wbeta'][l])
        out = beta * x_r + (1.0 - beta) * out
        h = _gelu(_ln(out + residual, p['ln_g'][l], p['ln_b'][l]))

    cnt = jnp.maximum(
        jax.ops.segment_sum(jnp.ones((N,), jnp.float32), batch, num_segments=B),
        1.0)[:, None]
    mean_pool = jax.ops.segment_sum(h, batch, num_segments=B) / cnt
    max_pool = jax.ops.segment_max(h, batch, num_segments=B)
    pooled = jnp.concatenate([mean_pool, max_pool], axis=-1)
    mu = pooled @ p['Wmu'] + p['bmu']
    logvar = pooled @ p['Wlv'] + p['blv']
    z = mu + eps * jnp.exp(0.5 * logvar)
    tok = p['tok_emb'][decoder_input_tokens]
    init = (z @ p['Wl2d'] + p['bl2d']).reshape(B, DEC_LAYERS, HID).transpose(1, 0, 2)
    xs = tok.transpose(1, 0, 2)
    for l in range(DEC_LAYERS):
        def step(hprev, x_t, Wih=p['Wih'][l], Whh=p['Whh'][l], bih=p['bih'][l],
                 bhh=p['bhh'][l]):
            gi = x_t @ Wih.T + bih
            gh = hprev @ Whh.T + bhh
            i_r, i_z, i_n = jnp.split(gi, 3, axis=-1)
            h_r, h_z, h_n = jnp.split(gh, 3, axis=-1)
            r = jax.nn.sigmoid(i_r + h_r)
            zg = jax.nn.sigmoid(i_z + h_z)
            n = jnp.tanh(i_n + r * h_n)
            hn = (1.0 - zg) * n + zg * hprev
            return hn, hn
        _, xs = jax.lax.scan(step, init[l], xs)
    out_seq = _ln(xs.transpose(1, 0, 2), p['dn_g'], p['dn_b'])
    token_logits = out_seq @ p['Wtok'] + p['btok']
    size_logits = _gelu(z @ p['Ws1'] + p['bs1']) @ p['Ws2'] + p['bs2']
    return (token_logits, size_logits, mu, logvar, z)


# speedup vs baseline: 2.1238x; 1.3532x over previous
import functools

import jax
import jax.numpy as jnp
import numpy as _np
from jax import lax
from jax.experimental import pallas as pl
from jax.experimental.pallas import tpu as pltpu
from jax.experimental.pallas import tpu_sc as plsc

N = 10000
E = 160000
B = 8
L = 128
D_IN = 12
HID = 512
EMB = 128
LAT = 256
N_LAYERS = 8
HEADS = 8
HEAD_DIM = 64
DEC_LAYERS = 3
N_TOKENS = 512
MAX_BLOCKS = 2048

ROW_BLK = 1000

PERM = _np.zeros(512, _np.int32)
for _w in range(32):
    for _l in range(16):
        _h = _l if _l < 8 else 15 - _l
        _d = 2 * _w if _l < 8 else 2 * _w + 1
        PERM[16 * _w + _l] = 64 * _h + _d

NC = 2
NS = 16
NW = NC * NS
BA = 64
RNG = 312
NTAIL = N - NW * RNG
SLAB = RNG + NTAIL


def _proj_body(h_ref, w_ref, b_ref, ep_ref, ev_ref, q_ref, k2_ref, va_ref,
               vb_ref, xr_ref):
    big = (jnp.dot(h_ref[...], w_ref[...], preferred_element_type=jnp.float32)
           + b_ref[...])
    q_ref[...] = big[:, 0:HID]
    k2_ref[...] = big[None, :, HID:2 * HID] + ep_ref[...][:, None, :]
    va_ref[...] = (big[None, :, 2 * HID:2 * HID + 256]
                   + ev_ref[...][:, None, 0:256])
    vb_ref[...] = (big[None, :, 2 * HID + 256:3 * HID]
                   + ev_ref[...][:, None, 256:512])
    xr_ref[...] = big[:, 3 * HID:4 * HID]


def _fused_proj(h, w, b, etabP, etabV):
    n, kdim = h.shape
    mdim = w.shape[1]
    grid = (n // ROW_BLK,)
    f32 = jnp.float32
    return pl.pallas_call(
        _proj_body,
        grid=grid,
        in_specs=[
            pl.BlockSpec((ROW_BLK, kdim), lambda i: (i, 0)),
            pl.BlockSpec((kdim, mdim), lambda i: (0, 0)),
            pl.BlockSpec((1, mdim), lambda i: (0, 0)),
            pl.BlockSpec((2, HID), lambda i: (0, 0)),
            pl.BlockSpec((2, HID), lambda i: (0, 0)),
        ],
        out_specs=[
            pl.BlockSpec((ROW_BLK, HID), lambda i: (i, 0)),
            pl.BlockSpec((2, ROW_BLK, HID), lambda i: (0, i, 0)),
            pl.BlockSpec((2, ROW_BLK, 256), lambda i: (0, i, 0)),
            pl.BlockSpec((2, ROW_BLK, 256), lambda i: (0, i, 0)),
            pl.BlockSpec((ROW_BLK, HID), lambda i: (i, 0)),
        ],
        out_shape=[
            jax.ShapeDtypeStruct((n, HID), f32),
            jax.ShapeDtypeStruct((2, n, HID), f32),
            jax.ShapeDtypeStruct((2, n, 256), f32),
            jax.ShapeDtypeStruct((2, n, 256), f32),
            jax.ShapeDtypeStruct((n, HID), f32),
        ],
    )(h, w, b.reshape(1, mdim), etabP, etabV)


def _worker_bounds(bndv, wid):
    lo = jnp.int32(0)
    hi = jnp.int32(0)
    for part in range(4):
        vec = bndv[pl.ds(16 * part, 16)]
        for i in range(16):
            idx = 16 * part + i
            if idx < 32:
                lo = jnp.where(wid == idx, vec[i], lo)
            else:
                hi = jnp.where(wid == (idx - 32), vec[i], hi)
    return lo, hi


def _edge_alpha_body(q_h, k2_h, src2_h, dst_h, bnd_h, ex_h, s_o,
                     srcv, dstv, bndv, qb, kb, exout, s_slab, sem):
    cid = lax.axis_index("c")
    sid = lax.axis_index("s")
    wid = sid * NC + cid
    node_lo = wid * RNG

    lane = lax.broadcasted_iota(jnp.int32, (16,), 0)
    lm8f = jnp.where(lane < 8, 1.0, 0.0).astype(jnp.float32)
    zeros16 = jnp.zeros((16,), jnp.float32)

    pltpu.sync_copy(bnd_h, bndv)
    lo_e, hi_e = _worker_bounds(bndv, wid)

    def zrow(r, c2):
        s_slab[r, :] = zeros16
        return c2

    lax.fori_loop(0, SLAB, zrow, 0)

    b_lo = lax.div(lo_e, BA)
    b_hi = lax.div(hi_e - 1, BA) + 1
    b_hi = jnp.maximum(b_hi, b_lo)

    def batch(bi, carry):
        base = bi * BA
        pltpu.sync_copy(dst_h.at[pl.ds(base, BA)], dstv)
        pltpu.sync_copy(src2_h.at[pl.ds(base, BA)], srcv)
        pltpu.async_copy(q_h.at[dstv], qb, sem).wait()
        pltpu.async_copy(k2_h.at[srcv], kb, sem).wait()

        def group(g, c3):
            dx = dstv[pl.ds(16 * g, 16)]
            for i2 in range(16):
                j = 16 * g + i2
                acc = qb[j, pl.ds(0, 16)] * kb[j, pl.ds(0, 16)]
                for w in range(1, 32):
                    acc = acc + (qb[j, pl.ds(16 * w, 16)]
                                 * kb[j, pl.ds(16 * w, 16)])
                raw = acc + jnp.flip(acc)
                exr = jnp.exp(raw * 0.125) * lm8f
                exout[j, :] = exr
                eidx = base + j
                valid = (eidx >= lo_e) & (eidx < hi_e)
                row = dx[i2] - node_lo

                @pl.when(valid)
                def _(row=row, exr=exr):
                    plsc.addupdate(s_slab.at[row], exr)
            return c3

        lax.fori_loop(0, BA // 16, group, 0)
        pltpu.sync_copy(exout, ex_h.at[pl.ds(base, BA)])
        return carry

    lax.fori_loop(b_lo, b_hi, batch, 0)

    pltpu.sync_copy(s_slab.at[pl.ds(0, RNG)], s_o.at[pl.ds(node_lo, RNG)])

    @pl.when(wid == NW - 1)
    def _():
        pltpu.sync_copy(s_slab.at[pl.ds(RNG, NTAIL)],
                        s_o.at[pl.ds(NW * RNG, NTAIL)])


def _edge_spread_body(va_h, vb_h, src2_h, dst_h, bnd_h, ex_h, acc_o,
                      srcv, dstv, bndv, vbuf, exv, acc_slab, sem):
    cid = lax.axis_index("c")
    sid = lax.axis_index("s")
    wid = sid * NC + cid
    node_lo = wid * RNG

    zeros16 = jnp.zeros((16,), jnp.float32)

    pltpu.sync_copy(bnd_h, bndv)
    lo_e, hi_e = _worker_bounds(bndv, wid)
    b_lo = lax.div(lo_e, BA)
    b_hi = lax.div(hi_e - 1, BA) + 1
    b_hi = jnp.maximum(b_hi, b_lo)

    for p, v_h in enumerate((va_h, vb_h)):

        def zrow(r, c2):
            for t in range(16):
                acc_slab[r, pl.ds(16 * t, 16)] = zeros16
            return c2

        lax.fori_loop(0, SLAB, zrow, 0)

        def batch(bi, carry, p=p, v_h=v_h):
            base = bi * BA
            pltpu.sync_copy(dst_h.at[pl.ds(base, BA)], dstv)
            pltpu.sync_copy(src2_h.at[pl.ds(base, BA)], srcv)
            pltpu.sync_copy(ex_h.at[pl.ds(base, BA)], exv)
            pltpu.async_copy(v_h.at[srcv], vbuf, sem).wait()

            def group(g, c3, p=p):
                dx = dstv[pl.ds(16 * g, 16)]
                for i2 in range(16):
                    j = 16 * g + i2
                    exrow = exv[j, :]
                    eidx = base + j
                    valid = (eidx >= lo_e) & (eidx < hi_e)
                    row = dx[i2] - node_lo

                    @pl.when(valid)
                    def _(row=row, exrow=exrow, j=j, p=p):
                        for t in range(16):
                            scale = exrow[4 * p + t // 4]
                            plsc.addupdate(
                                acc_slab.at[row, pl.ds(16 * t, 16)],
                                vbuf[j, pl.ds(16 * t, 16)] * scale)
                return c3

            lax.fori_loop(0, BA // 16, group, 0)
            return carry

        lax.fori_loop(b_lo, b_hi, batch, 0)

        pltpu.sync_copy(acc_slab.at[pl.ds(0, RNG)],
                        acc_o.at[p, pl.ds(node_lo, RNG)])

        @pl.when(wid == NW - 1)
        def _(p=p):
            pltpu.sync_copy(acc_slab.at[pl.ds(RNG, NTAIL)],
                            acc_o.at[p, pl.ds(NW * RNG, NTAIL)])


def _edge_aggregate(qp, k2, va, vb, src2, dsts, bnd):
    mesh = plsc.VectorSubcoreMesh(core_axis_name="c", subcore_axis_name="s",
                                  num_cores=NC, num_subcores=NS)
    f32 = jnp.float32
    run_a = pl.kernel(
        _edge_alpha_body,
        out_type=(
            jax.ShapeDtypeStruct((E, 16), f32),
            jax.ShapeDtypeStruct((N, 16), f32),
        ),
        mesh=mesh,
        scratch_types=[
            pltpu.VMEM((BA,), jnp.int32),
            pltpu.VMEM((BA,), jnp.int32),
            pltpu.VMEM((64,), jnp.int32),
            pltpu.VMEM((BA, HID), f32),
            pltpu.VMEM((BA, HID), f32),
            pltpu.VMEM((BA, 16), f32),
            pltpu.VMEM((SLAB, 16), f32),
            pltpu.SemaphoreType.DMA,
        ],
    )
    ex, s = run_a(qp, k2, src2, dsts, bnd)
    run_b = pl.kernel(
        _edge_spread_body,
        out_type=jax.ShapeDtypeStruct((2, N, 256), f32),
        mesh=mesh,
        scratch_types=[
            pltpu.VMEM((BA,), jnp.int32),
            pltpu.VMEM((BA,), jnp.int32),
            pltpu.VMEM((64,), jnp.int32),
            pltpu.VMEM((BA, 256), f32),
            pltpu.VMEM((BA, 16), f32),
            pltpu.VMEM((SLAB, 256), f32),
            pltpu.SemaphoreType.DMA,
        ],
    )
    acc = run_b(va, vb, src2, dsts, bnd, ex)
    return ex, s, acc


def _ln(x, g, b):
    m = jnp.mean(x, -1, keepdims=True)
    v = jnp.mean((x - m) ** 2, -1, keepdims=True)
    return g * (x - m) / jnp.sqrt(v + 1e-5) + b


def _gelu(x):
    return jax.nn.gelu(x, approximate=False)


def kernel(x, node_type, block_data, port_direction, edge_type, edge_index,
           batch, decoder_input_tokens, eps, params):
    p = params
    nt = jnp.remainder(jnp.maximum(node_type, 0), 2)
    bd = jnp.remainder(jnp.maximum(block_data, 0), 64)
    pd = jnp.remainder(jnp.maximum(port_direction + 1, 0), 8)
    et = jnp.remainder(jnp.maximum(edge_type, 0), 2).astype(jnp.int32)
    feats = jnp.concatenate(
        [x.astype(jnp.float32), p['node_type_emb'][nt], p['block_data_emb'][bd],
         p['port_dir_emb'][pd]], axis=-1)
    h = feats @ p['Wp'] + p['bp']
    src = edge_index[0].astype(jnp.int32)
    dst = edge_index[1].astype(jnp.int32)

    order = jnp.argsort(dst)
    dsts = dst[order]
    srcs = src[order]
    ets = et[order]
    src2 = srcs + ets * N
    cuts = jnp.searchsorted(dsts, (jnp.arange(1, NW) * RNG).astype(jnp.int32))
    bnd = jnp.concatenate([
        jnp.zeros((1,), jnp.int32), cuts.astype(jnp.int32),
        cuts.astype(jnp.int32), jnp.full((1,), E, jnp.int32)]).astype(jnp.int32)

    perm = jnp.asarray(PERM)

    for l in range(N_LAYERS):
        residual = h
        w_all = jnp.concatenate(
            [p['Wq'][l][:, perm], p['Wk'][l][:, perm], p['Wv'][l],
             p['Wskip'][l]], axis=1)
        b_all = jnp.concatenate(
            [p['bq'][l][perm], p['bk'][l][perm], p['bv'][l], p['bskip'][l]],
            axis=0)
        etab = p['edge_type_emb'] @ p['We'][l] + p['be'][l]
        etabP = etab[:, perm]

        qp, k2, va, vb, x_r = _fused_proj(h, w_all, b_all, etabP, etab)
        ex, s, acc = _edge_aggregate(
            qp, k2.reshape(2 * N, HID), va.reshape(2 * N, 256),
            vb.reshape(2 * N, 256), src2, dsts, bnd)

        s8 = s[:, 0:8]
        num = jnp.concatenate([acc[0], acc[1]], axis=-1)
        out = num / (jnp.repeat(s8, HEAD_DIM, axis=1) + 1e-16)

        beta = jax.nn.sigmoid(
            jnp.concatenate([out, x_r, out - x_r], axis=-1) @ p['Wbeta'][l])
        out = beta * x_r + (1.0 - beta) * out
        h = _gelu(_ln(out + residual, p['ln_g'][l], p['ln_b'][l]))

    cnt = jnp.maximum(
        jax.ops.segment_sum(jnp.ones((N,), jnp.float32), batch, num_segments=B),
        1.0)[:, None]
    mean_pool = jax.ops.segment_sum(h, batch, num_segments=B) / cnt
    max_pool = jax.ops.segment_max(h, batch, num_segments=B)
    pooled = jnp.concatenate([mean_pool, max_pool], axis=-1)
    mu = pooled @ p['Wmu'] + p['bmu']
    logvar = pooled @ p['Wlv'] + p['blv']
    z = mu + eps * jnp.exp(0.5 * logvar)
    tok = p['tok_emb'][decoder_input_tokens]
    init = (z @ p['Wl2d'] + p['bl2d']).reshape(B, DEC_LAYERS, HID).transpose(1, 0, 2)
    xs = tok.transpose(1, 0, 2)
    for l in range(DEC_LAYERS):
        def step(hprev, x_t, Wih=p['Wih'][l], Whh=p['Whh'][l], bih=p['bih'][l],
                 bhh=p['bhh'][l]):
            gi = x_t @ Wih.T + bih
            gh = hprev @ Whh.T + bhh
            i_r, i_z, i_n = jnp.split(gi, 3, axis=-1)
            h_r, h_z, h_n = jnp.split(gh, 3, axis=-1)
            r = jax.nn.sigmoid(i_r + h_r)
            zg = jax.nn.sigmoid(i_z + h_z)
            n = jnp.tanh(i_n + r * h_n)
            hn = (1.0 - zg) * n + zg * hprev
            return hn, hn
        _, xs = jax.lax.scan(step, init[l], xs)
    out_seq = _ln(xs.transpose(1, 0, 2), p['dn_g'], p['dn_b'])
    token_logits = out_seq @ p['Wtok'] + p['btok']
    size_logits = _gelu(z @ p['Ws1'] + p['bs1']) @ p['Ws2'] + p['bs2']
    return (token_logits, size_logits, mu, logvar, z)


# speedup vs baseline: 2.3268x; 1.0956x over previous
import functools

import jax
import jax.numpy as jnp
import numpy as _np
from jax import lax
from jax.experimental import pallas as pl
from jax.experimental.pallas import tpu as pltpu
from jax.experimental.pallas import tpu_sc as plsc

N = 10000
E = 160000
B = 8
L = 128
D_IN = 12
HID = 512
EMB = 128
LAT = 256
N_LAYERS = 8
HEADS = 8
HEAD_DIM = 64
DEC_LAYERS = 3
N_TOKENS = 512
MAX_BLOCKS = 2048

ROW_BLK = 1000

PERM = _np.zeros(512, _np.int32)
for _w in range(32):
    for _l in range(16):
        _h = _l if _l < 8 else 15 - _l
        _d = 2 * _w if _l < 8 else 2 * _w + 1
        PERM[16 * _w + _l] = 64 * _h + _d

NC = 2
NS = 16
NW = NC * NS
BA = 64
RNG = 312
NTAIL = N - NW * RNG
SLAB = RNG + NTAIL


def _proj_body(h_ref, w_ref, b_ref, ep_ref, ev_ref, q_ref, k2_ref, va_ref,
               vb_ref, xr_ref):
    big = (jnp.dot(h_ref[...], w_ref[...], preferred_element_type=jnp.float32)
           + b_ref[...])
    q_ref[...] = big[:, 0:HID]
    k2_ref[...] = big[None, :, HID:2 * HID] + ep_ref[...][:, None, :]
    va_ref[...] = (big[None, :, 2 * HID:2 * HID + 256]
                   + ev_ref[...][:, None, 0:256])
    vb_ref[...] = (big[None, :, 2 * HID + 256:3 * HID]
                   + ev_ref[...][:, None, 256:512])
    xr_ref[...] = big[:, 3 * HID:4 * HID]


def _fused_proj(h, w, b, etabP, etabV):
    n, kdim = h.shape
    mdim = w.shape[1]
    grid = (n // ROW_BLK,)
    f32 = jnp.float32
    return pl.pallas_call(
        _proj_body,
        grid=grid,
        in_specs=[
            pl.BlockSpec((ROW_BLK, kdim), lambda i: (i, 0)),
            pl.BlockSpec((kdim, mdim), lambda i: (0, 0)),
            pl.BlockSpec((1, mdim), lambda i: (0, 0)),
            pl.BlockSpec((2, HID), lambda i: (0, 0)),
            pl.BlockSpec((2, HID), lambda i: (0, 0)),
        ],
        out_specs=[
            pl.BlockSpec((ROW_BLK, HID), lambda i: (i, 0)),
            pl.BlockSpec((2, ROW_BLK, HID), lambda i: (0, i, 0)),
            pl.BlockSpec((2, ROW_BLK, 256), lambda i: (0, i, 0)),
            pl.BlockSpec((2, ROW_BLK, 256), lambda i: (0, i, 0)),
            pl.BlockSpec((ROW_BLK, HID), lambda i: (i, 0)),
        ],
        out_shape=[
            jax.ShapeDtypeStruct((n, HID), f32),
            jax.ShapeDtypeStruct((2, n, HID), f32),
            jax.ShapeDtypeStruct((2, n, 256), f32),
            jax.ShapeDtypeStruct((2, n, 256), f32),
            jax.ShapeDtypeStruct((n, HID), f32),
        ],
    )(h, w, b.reshape(1, mdim), etabP, etabV)


def _worker_bounds(bndv, wid):
    lo = jnp.int32(0)
    hi = jnp.int32(0)
    for part in range(4):
        vec = bndv[pl.ds(16 * part, 16)]
        for i in range(16):
            idx = 16 * part + i
            if idx < 32:
                lo = jnp.where(wid == idx, vec[i], lo)
            else:
                hi = jnp.where(wid == (idx - 32), vec[i], hi)
    return lo, hi


def _edge_alpha_body(q_h, k2_h, src2_h, dst_h, bnd_h, ex_h, s_o,
                     srcv, dstv, bndv, qb, kb, exout, s_slab, sem):
    cid = lax.axis_index("c")
    sid = lax.axis_index("s")
    wid = sid * NC + cid
    node_lo = wid * RNG

    lane = lax.broadcasted_iota(jnp.int32, (16,), 0)
    lm8f = jnp.where(lane < 8, 1.0, 0.0).astype(jnp.float32)
    zeros16 = jnp.zeros((16,), jnp.float32)

    pltpu.sync_copy(bnd_h, bndv)
    lo_e, hi_e = _worker_bounds(bndv, wid)

    def zrow(r, c2):
        s_slab[r, :] = zeros16
        return c2

    lax.fori_loop(0, SLAB, zrow, 0)

    b_lo = lax.div(lo_e, BA)
    b_hi = lax.div(hi_e - 1, BA) + 1
    b_hi = jnp.maximum(b_hi, b_lo)

    def batch(bi, carry):
        base = bi * BA
        c1 = pltpu.async_copy(dst_h.at[pl.ds(base, BA)], dstv, sem)
        c2 = pltpu.async_copy(src2_h.at[pl.ds(base, BA)], srcv, sem)
        c1.wait()
        c2.wait()
        g1 = pltpu.async_copy(q_h.at[dstv], qb, sem)
        g2 = pltpu.async_copy(k2_h.at[srcv], kb, sem)
        g1.wait()
        g2.wait()

        def group(g, c3):
            dx = dstv[pl.ds(16 * g, 16)]
            for i2 in range(16):
                j = 16 * g + i2
                acc = qb[j, pl.ds(0, 16)] * kb[j, pl.ds(0, 16)]
                for w in range(1, 32):
                    acc = acc + (qb[j, pl.ds(16 * w, 16)]
                                 * kb[j, pl.ds(16 * w, 16)])
                raw = acc + jnp.flip(acc)
                exr = jnp.exp(raw * 0.125) * lm8f
                exout[j, :] = exr
                eidx = base + j
                valid = (eidx >= lo_e) & (eidx < hi_e)
                row = dx[i2] - node_lo

                @pl.when(valid)
                def _(row=row, exr=exr):
                    plsc.addupdate(s_slab.at[row], exr)
            return c3

        lax.fori_loop(0, BA // 16, group, 0)
        pltpu.sync_copy(exout, ex_h.at[pl.ds(base, BA)])
        return carry

    lax.fori_loop(b_lo, b_hi, batch, 0)

    pltpu.sync_copy(s_slab.at[pl.ds(0, RNG)], s_o.at[pl.ds(node_lo, RNG)])

    @pl.when(wid == NW - 1)
    def _():
        pltpu.sync_copy(s_slab.at[pl.ds(RNG, NTAIL)],
                        s_o.at[pl.ds(NW * RNG, NTAIL)])


def _edge_spread_body(va_h, vb_h, src2_h, dst_h, bnd_h, ex_h, acc_o,
                      srcv, dstv, bndv, vbuf, exv, acc_slab, sem):
    cid = lax.axis_index("c")
    sid = lax.axis_index("s")
    wid = sid * NC + cid
    node_lo = wid * RNG

    zeros16 = jnp.zeros((16,), jnp.float32)

    pltpu.sync_copy(bnd_h, bndv)
    lo_e, hi_e = _worker_bounds(bndv, wid)
    b_lo = lax.div(lo_e, BA)
    b_hi = lax.div(hi_e - 1, BA) + 1
    b_hi = jnp.maximum(b_hi, b_lo)

    for p, v_h in enumerate((va_h, vb_h)):

        def zrow(r, c2):
            for t in range(16):
                acc_slab[r, pl.ds(16 * t, 16)] = zeros16
            return c2

        lax.fori_loop(0, SLAB, zrow, 0)

        def batch(bi, carry, p=p, v_h=v_h):
            base = bi * BA
            c1 = pltpu.async_copy(dst_h.at[pl.ds(base, BA)], dstv, sem)
            c2 = pltpu.async_copy(src2_h.at[pl.ds(base, BA)], srcv, sem)
            c3 = pltpu.async_copy(ex_h.at[pl.ds(base, BA)], exv, sem)
            c1.wait()
            c2.wait()
            c3.wait()
            pltpu.async_copy(v_h.at[srcv], vbuf, sem).wait()

            def group(g, c3, p=p):
                dx = dstv[pl.ds(16 * g, 16)]
                for i2 in range(16):
                    j = 16 * g + i2
                    exrow = exv[j, :]
                    eidx = base + j
                    valid = (eidx >= lo_e) & (eidx < hi_e)
                    row = dx[i2] - node_lo

                    @pl.when(valid)
                    def _(row=row, exrow=exrow, j=j, p=p):
                        for t in range(16):
                            scale = exrow[4 * p + t // 4]
                            plsc.addupdate(
                                acc_slab.at[row, pl.ds(16 * t, 16)],
                                vbuf[j, pl.ds(16 * t, 16)] * scale)
                return c3

            lax.fori_loop(0, BA // 16, group, 0)
            return carry

        lax.fori_loop(b_lo, b_hi, batch, 0)

        pltpu.sync_copy(acc_slab.at[pl.ds(0, RNG)],
                        acc_o.at[p, pl.ds(node_lo, RNG)])

        @pl.when(wid == NW - 1)
        def _(p=p):
            pltpu.sync_copy(acc_slab.at[pl.ds(RNG, NTAIL)],
                            acc_o.at[p, pl.ds(NW * RNG, NTAIL)])


def _edge_aggregate(qp, k2, va, vb, src2, dsts, bnd):
    mesh = plsc.VectorSubcoreMesh(core_axis_name="c", subcore_axis_name="s",
                                  num_cores=NC, num_subcores=NS)
    f32 = jnp.float32
    run_a = pl.kernel(
        _edge_alpha_body,
        out_type=(
            jax.ShapeDtypeStruct((E, 16), f32),
            jax.ShapeDtypeStruct((N, 16), f32),
        ),
        mesh=mesh,
        scratch_types=[
            pltpu.VMEM((BA,), jnp.int32),
            pltpu.VMEM((BA,), jnp.int32),
            pltpu.VMEM((64,), jnp.int32),
            pltpu.VMEM((BA, HID), f32),
            pltpu.VMEM((BA, HID), f32),
            pltpu.VMEM((BA, 16), f32),
            pltpu.VMEM((SLAB, 16), f32),
            pltpu.SemaphoreType.DMA,
        ],
    )
    ex, s = run_a(qp, k2, src2, dsts, bnd)
    run_b = pl.kernel(
        _edge_spread_body,
        out_type=jax.ShapeDtypeStruct((2, N, 256), f32),
        mesh=mesh,
        scratch_types=[
            pltpu.VMEM((BA,), jnp.int32),
            pltpu.VMEM((BA,), jnp.int32),
            pltpu.VMEM((64,), jnp.int32),
            pltpu.VMEM((BA, 256), f32),
            pltpu.VMEM((BA, 16), f32),
            pltpu.VMEM((SLAB, 256), f32),
            pltpu.SemaphoreType.DMA,
        ],
    )
    acc = run_b(va, vb, src2, dsts, bnd, ex)
    return ex, s, acc


def _ln(x, g, b):
    m = jnp.mean(x, -1, keepdims=True)
    v = jnp.mean((x - m) ** 2, -1, keepdims=True)
    return g * (x - m) / jnp.sqrt(v + 1e-5) + b


def _gelu(x):
    return jax.nn.gelu(x, approximate=False)


def kernel(x, node_type, block_data, port_direction, edge_type, edge_index,
           batch, decoder_input_tokens, eps, params):
    p = params
    nt = jnp.remainder(jnp.maximum(node_type, 0), 2)
    bd = jnp.remainder(jnp.maximum(block_data, 0), 64)
    pd = jnp.remainder(jnp.maximum(port_direction + 1, 0), 8)
    et = jnp.remainder(jnp.maximum(edge_type, 0), 2).astype(jnp.int32)
    feats = jnp.concatenate(
        [x.astype(jnp.float32), p['node_type_emb'][nt], p['block_data_emb'][bd],
         p['port_dir_emb'][pd]], axis=-1)
    h = feats @ p['Wp'] + p['bp']
    src = edge_index[0].astype(jnp.int32)
    dst = edge_index[1].astype(jnp.int32)

    order = jnp.argsort(dst)
    dsts = dst[order]
    srcs = src[order]
    ets = et[order]
    src2 = srcs + ets * N
    cuts = jnp.searchsorted(dsts, (jnp.arange(1, NW) * RNG).astype(jnp.int32))
    bnd = jnp.concatenate([
        jnp.zeros((1,), jnp.int32), cuts.astype(jnp.int32),
        cuts.astype(jnp.int32), jnp.full((1,), E, jnp.int32)]).astype(jnp.int32)

    perm = jnp.asarray(PERM)

    for l in range(N_LAYERS):
        residual = h
        w_all = jnp.concatenate(
            [p['Wq'][l][:, perm], p['Wk'][l][:, perm], p['Wv'][l],
             p['Wskip'][l]], axis=1)
        b_all = jnp.concatenate(
            [p['bq'][l][perm], p['bk'][l][perm], p['bv'][l], p['bskip'][l]],
            axis=0)
        etab = p['edge_type_emb'] @ p['We'][l] + p['be'][l]
        etabP = etab[:, perm]

        qp, k2, va, vb, x_r = _fused_proj(h, w_all, b_all, etabP, etab)
        ex, s, acc = _edge_aggregate(
            qp, k2.reshape(2 * N, HID), va.reshape(2 * N, 256),
            vb.reshape(2 * N, 256), src2, dsts, bnd)

        s8 = s[:, 0:8]
        num = jnp.concatenate([acc[0], acc[1]], axis=-1)
        out = num / (jnp.repeat(s8, HEAD_DIM, axis=1) + 1e-16)

        beta = jax.nn.sigmoid(
            jnp.concatenate([out, x_r, out - x_r], axis=-1) @ p['Wbeta'][l])
        out = beta * x_r + (1.0 - beta) * out
        h = _gelu(_ln(out + residual, p['ln_g'][l], p['ln_b'][l]))

    cnt = jnp.maximum(
        jax.ops.segment_sum(jnp.ones((N,), jnp.float32), batch, num_segments=B),
        1.0)[:, None]
    mean_pool = jax.ops.segment_sum(h, batch, num_segments=B) / cnt
    max_pool = jax.ops.segment_max(h, batch, num_segments=B)
    pooled = jnp.concatenate([mean_pool, max_pool], axis=-1)
    mu = pooled @ p['Wmu'] + p['bmu']
    logvar = pooled @ p['Wlv'] + p['blv']
    z = mu + eps * jnp.exp(0.5 * logvar)
    tok = p['tok_emb'][decoder_input_tokens]
    init = (z @ p['Wl2d'] + p['bl2d']).reshape(B, DEC_LAYERS, HID).transpose(1, 0, 2)
    xs = tok.transpose(1, 0, 2)
    for l in range(DEC_LAYERS):
        def step(hprev, x_t, Wih=p['Wih'][l], Whh=p['Whh'][l], bih=p['bih'][l],
                 bhh=p['bhh'][l]):
            gi = x_t @ Wih.T + bih
            gh = hprev @ Whh.T + bhh
            i_r, i_z, i_n = jnp.split(gi, 3, axis=-1)
            h_r, h_z, h_n = jnp.split(gh, 3, axis=-1)
            r = jax.nn.sigmoid(i_r + h_r)
            zg = jax.nn.sigmoid(i_z + h_z)
            n = jnp.tanh(i_n + r * h_n)
            hn = (1.0 - zg) * n + zg * hprev
            return hn, hn
        _, xs = jax.lax.scan(step, init[l], xs)
    out_seq = _ln(xs.transpose(1, 0, 2), p['dn_g'], p['dn_b'])
    token_logits = out_seq @ p['Wtok'] + p['btok']
    size_logits = _gelu(z @ p['Ws1'] + p['bs1']) @ p['Ws2'] + p['bs2']
    return (token_logits, size_logits, mu, logvar, z)


# speedup vs baseline: 2.3634x; 1.0157x over previous
import functools

import jax
import jax.numpy as jnp
import numpy as _np
from jax import lax
from jax.experimental import pallas as pl
from jax.experimental.pallas import tpu as pltpu
from jax.experimental.pallas import tpu_sc as plsc

N = 10000
E = 160000
B = 8
L = 128
D_IN = 12
HID = 512
EMB = 128
LAT = 256
N_LAYERS = 8
HEADS = 8
HEAD_DIM = 64
DEC_LAYERS = 3
N_TOKENS = 512
MAX_BLOCKS = 2048

ROW_BLK = 1000

PERM = _np.zeros(512, _np.int32)
for _w in range(32):
    for _l in range(16):
        _h = _l if _l < 8 else 15 - _l
        _d = 2 * _w if _l < 8 else 2 * _w + 1
        PERM[16 * _w + _l] = 64 * _h + _d

NC = 2
NS = 16
NW = NC * NS
BA = 64
RNG = 312
NTAIL = N - NW * RNG
SLAB = RNG + NTAIL


def _proj_body(h_ref, w_ref, b_ref, ep_ref, ev_ref, q_ref, k2_ref, va_ref,
               vb_ref, xr_ref):
    big = (jnp.dot(h_ref[...], w_ref[...], preferred_element_type=jnp.float32)
           + b_ref[...])
    q_ref[...] = big[:, 0:HID]
    k2_ref[...] = big[None, :, HID:2 * HID] + ep_ref[...][:, None, :]
    va_ref[...] = (big[None, :, 2 * HID:2 * HID + 256]
                   + ev_ref[...][:, None, 0:256])
    vb_ref[...] = (big[None, :, 2 * HID + 256:3 * HID]
                   + ev_ref[...][:, None, 256:512])
    xr_ref[...] = big[:, 3 * HID:4 * HID]


def _fused_proj(h, w, b, etabP, etabV):
    n, kdim = h.shape
    mdim = w.shape[1]
    grid = (n // ROW_BLK,)
    f32 = jnp.float32
    return pl.pallas_call(
        _proj_body,
        grid=grid,
        in_specs=[
            pl.BlockSpec((ROW_BLK, kdim), lambda i: (i, 0)),
            pl.BlockSpec((kdim, mdim), lambda i: (0, 0)),
            pl.BlockSpec((1, mdim), lambda i: (0, 0)),
            pl.BlockSpec((2, HID), lambda i: (0, 0)),
            pl.BlockSpec((2, HID), lambda i: (0, 0)),
        ],
        out_specs=[
            pl.BlockSpec((ROW_BLK, HID), lambda i: (i, 0)),
            pl.BlockSpec((2, ROW_BLK, HID), lambda i: (0, i, 0)),
            pl.BlockSpec((2, ROW_BLK, 256), lambda i: (0, i, 0)),
            pl.BlockSpec((2, ROW_BLK, 256), lambda i: (0, i, 0)),
            pl.BlockSpec((ROW_BLK, HID), lambda i: (i, 0)),
        ],
        out_shape=[
            jax.ShapeDtypeStruct((n, HID), f32),
            jax.ShapeDtypeStruct((2, n, HID), f32),
            jax.ShapeDtypeStruct((2, n, 256), f32),
            jax.ShapeDtypeStruct((2, n, 256), f32),
            jax.ShapeDtypeStruct((n, HID), f32),
        ],
    )(h, w, b.reshape(1, mdim), etabP, etabV)


def _worker_bounds(bndv, wid):
    lo = jnp.int32(0)
    hi = jnp.int32(0)
    for part in range(4):
        vec = bndv[pl.ds(16 * part, 16)]
        for i in range(16):
            idx = 16 * part + i
            if idx < 32:
                lo = jnp.where(wid == idx, vec[i], lo)
            else:
                hi = jnp.where(wid == (idx - 32), vec[i], hi)
    return lo, hi


def _edge_alpha_body(q_h, k2_h, src2_h, dst_h, bnd_h, ex_h, s_o,
                     srcv, dstv, bndv, qb, kb, exout, s_slab, sem):
    cid = lax.axis_index("c")
    sid = lax.axis_index("s")
    wid = sid * NC + cid
    node_lo = wid * RNG

    lane = lax.broadcasted_iota(jnp.int32, (16,), 0)
    lm8f = jnp.where(lane < 8, 1.0, 0.0).astype(jnp.float32)
    zeros16 = jnp.zeros((16,), jnp.float32)

    pltpu.sync_copy(bnd_h, bndv)
    lo_e, hi_e = _worker_bounds(bndv, wid)

    def zrow(r, c2):
        s_slab[r, :] = zeros16
        return c2

    lax.fori_loop(0, SLAB, zrow, 0)

    b_lo = lax.div(lo_e, BA)
    b_hi = lax.div(hi_e - 1, BA) + 1
    b_hi = jnp.maximum(b_hi, b_lo)

    def batch(bi, carry):
        base = bi * BA
        c1 = pltpu.async_copy(dst_h.at[pl.ds(base, BA)], dstv, sem)
        c2 = pltpu.async_copy(src2_h.at[pl.ds(base, BA)], srcv, sem)
        c1.wait()
        c2.wait()
        g1 = pltpu.async_copy(q_h.at[dstv], qb, sem)
        g2 = pltpu.async_copy(k2_h.at[srcv], kb, sem)
        g1.wait()
        g2.wait()

        def group(g, c3):
            dx = dstv[pl.ds(16 * g, 16)]
            for i2 in range(16):
                j = 16 * g + i2
                acc = qb[j, pl.ds(0, 16)] * kb[j, pl.ds(0, 16)]
                for w in range(1, 32):
                    acc = acc + (qb[j, pl.ds(16 * w, 16)]
                                 * kb[j, pl.ds(16 * w, 16)])
                raw = acc + jnp.flip(acc)
                exr = jnp.exp(raw * 0.125) * lm8f
                exout[j, :] = exr
                eidx = base + j
                valid = (eidx >= lo_e) & (eidx < hi_e)
                row = dx[i2] - node_lo

                @pl.when(valid)
                def _(row=row, exr=exr):
                    plsc.addupdate(s_slab.at[row], exr)
            return c3

        lax.fori_loop(0, BA // 16, group, 0)
        pltpu.sync_copy(exout, ex_h.at[pl.ds(base, BA)])
        return carry

    lax.fori_loop(b_lo, b_hi, batch, 0)

    pltpu.sync_copy(s_slab.at[pl.ds(0, RNG)], s_o.at[pl.ds(node_lo, RNG)])

    @pl.when(wid == NW - 1)
    def _():
        pltpu.sync_copy(s_slab.at[pl.ds(RNG, NTAIL)],
                        s_o.at[pl.ds(NW * RNG, NTAIL)])


def _edge_spread_body(va_h, vb_h, src2_h, dst_h, bnd_h, ex_h, acc_o,
                      srcv, dstv, bndv, vbuf, exv, acc_slab, sem):
    cid = lax.axis_index("c")
    sid = lax.axis_index("s")
    wid = sid * NC + cid
    node_lo = wid * RNG

    zeros16 = jnp.zeros((16,), jnp.float32)

    pltpu.sync_copy(bnd_h, bndv)
    lo_e, hi_e = _worker_bounds(bndv, wid)
    b_lo = lax.div(lo_e, BA)
    b_hi = lax.div(hi_e - 1, BA) + 1
    b_hi = jnp.maximum(b_hi, b_lo)

    for p, v_h in enumerate((va_h, vb_h)):

        def zrow(r, c2):
            for t in range(16):
                acc_slab[r, pl.ds(16 * t, 16)] = zeros16
            return c2

        lax.fori_loop(0, SLAB, zrow, 0)

        def batch(bi, carry, p=p, v_h=v_h):
            base = bi * BA
            c1 = pltpu.async_copy(dst_h.at[pl.ds(base, BA)], dstv, sem)
            c2 = pltpu.async_copy(src2_h.at[pl.ds(base, BA)], srcv, sem)
            c3 = pltpu.async_copy(ex_h.at[pl.ds(base, BA)], exv, sem)
            c1.wait()
            c2.wait()
            c3.wait()
            pltpu.async_copy(v_h.at[srcv], vbuf, sem).wait()

            def group(g, c3, p=p):
                dx = dstv[pl.ds(16 * g, 16)]
                for i2 in range(16):
                    j = 16 * g + i2
                    exrow = exv[j, :]
                    eidx = base + j
                    valid = (eidx >= lo_e) & (eidx < hi_e)
                    row = dx[i2] - node_lo

                    @pl.when(valid)
                    def _(row=row, exrow=exrow, j=j, p=p):
                        for t in range(16):
                            scale = exrow[4 * p + t // 4]
                            plsc.addupdate(
                                acc_slab.at[row, pl.ds(16 * t, 16)],
                                vbuf[j, pl.ds(16 * t, 16)] * scale)
                return c3

            lax.fori_loop(0, BA // 16, group, 0)
            return carry

        lax.fori_loop(b_lo, b_hi, batch, 0)

        pltpu.sync_copy(acc_slab.at[pl.ds(0, RNG)],
                        acc_o.at[p, pl.ds(node_lo, RNG)])

        @pl.when(wid == NW - 1)
        def _(p=p):
            pltpu.sync_copy(acc_slab.at[pl.ds(RNG, NTAIL)],
                            acc_o.at[p, pl.ds(NW * RNG, NTAIL)])


def _edge_aggregate(qp, k2, va, vb, src2, dsts, bnd):
    mesh = plsc.VectorSubcoreMesh(core_axis_name="c", subcore_axis_name="s",
                                  num_cores=NC, num_subcores=NS)
    f32 = jnp.float32
    run_a = pl.kernel(
        _edge_alpha_body,
        out_type=(
            jax.ShapeDtypeStruct((E, 16), f32),
            jax.ShapeDtypeStruct((N, 16), f32),
        ),
        mesh=mesh,
        scratch_types=[
            pltpu.VMEM((BA,), jnp.int32),
            pltpu.VMEM((BA,), jnp.int32),
            pltpu.VMEM((64,), jnp.int32),
            pltpu.VMEM((BA, HID), f32),
            pltpu.VMEM((BA, HID), f32),
            pltpu.VMEM((BA, 16), f32),
            pltpu.VMEM((SLAB, 16), f32),
            pltpu.SemaphoreType.DMA,
        ],
    )
    ex, s = run_a(qp, k2, src2, dsts, bnd)
    run_b = pl.kernel(
        _edge_spread_body,
        out_type=jax.ShapeDtypeStruct((2, N, 256), f32),
        mesh=mesh,
        scratch_types=[
            pltpu.VMEM((BA,), jnp.int32),
            pltpu.VMEM((BA,), jnp.int32),
            pltpu.VMEM((64,), jnp.int32),
            pltpu.VMEM((BA, 256), f32),
            pltpu.VMEM((BA, 16), f32),
            pltpu.VMEM((SLAB, 256), f32),
            pltpu.SemaphoreType.DMA,
        ],
    )
    acc = run_b(va, vb, src2, dsts, bnd, ex)
    return ex, s, acc


def _ln(x, g, b):
    m = jnp.mean(x, -1, keepdims=True)
    v = jnp.mean((x - m) ** 2, -1, keepdims=True)
    return g * (x - m) / jnp.sqrt(v + 1e-5) + b


def _gelu(x):
    return jax.nn.gelu(x, approximate=False)


def kernel(x, node_type, block_data, port_direction, edge_type, edge_index,
           batch, decoder_input_tokens, eps, params):
    p = params
    nt = jnp.remainder(jnp.maximum(node_type, 0), 2)
    bd = jnp.remainder(jnp.maximum(block_data, 0), 64)
    pd = jnp.remainder(jnp.maximum(port_direction + 1, 0), 8)
    et = jnp.remainder(jnp.maximum(edge_type, 0), 2).astype(jnp.int32)
    feats = jnp.concatenate(
        [x.astype(jnp.float32), p['node_type_emb'][nt], p['block_data_emb'][bd],
         p['port_dir_emb'][pd]], axis=-1)
    h = feats @ p['Wp'] + p['bp']
    src = edge_index[0].astype(jnp.int32)
    dst = edge_index[1].astype(jnp.int32)

    order = jnp.argsort(dst)
    dsts = dst[order]
    srcs = src[order]
    ets = et[order]
    src2 = srcs + ets * N
    cuts = jnp.searchsorted(dsts, (jnp.arange(1, NW) * RNG).astype(jnp.int32))
    bnd = jnp.concatenate([
        jnp.zeros((1,), jnp.int32), cuts.astype(jnp.int32),
        cuts.astype(jnp.int32), jnp.full((1,), E, jnp.int32)]).astype(jnp.int32)

    perm = jnp.asarray(PERM)

    for l in range(N_LAYERS):
        residual = h
        w_all = jnp.concatenate(
            [p['Wq'][l][:, perm], p['Wk'][l][:, perm], p['Wv'][l],
             p['Wskip'][l]], axis=1)
        b_all = jnp.concatenate(
            [p['bq'][l][perm], p['bk'][l][perm], p['bv'][l], p['bskip'][l]],
            axis=0)
        etab = p['edge_type_emb'] @ p['We'][l] + p['be'][l]
        etabP = etab[:, perm]

        qp, k2, va, vb, x_r = _fused_proj(h, w_all, b_all, etabP, etab)
        ex, s, acc = _edge_aggregate(
            qp, k2.reshape(2 * N, HID), va.reshape(2 * N, 256),
            vb.reshape(2 * N, 256), src2, dsts, bnd)

        s8 = s[:, 0:8]
        num = jnp.concatenate([acc[0], acc[1]], axis=-1)
        out = num / (jnp.repeat(s8, HEAD_DIM, axis=1) + 1e-16)

        beta = jax.nn.sigmoid(
            jnp.concatenate([out, x_r, out - x_r], axis=-1) @ p['Wbeta'][l])
        out = beta * x_r + (1.0 - beta) * out
        h = _gelu(_ln(out + residual, p['ln_g'][l], p['ln_b'][l]))

    oh = (batch[:, None] == jnp.arange(B)[None, :]).astype(jnp.float32)
    cnt = jnp.maximum(jnp.sum(oh, axis=0), 1.0)[:, None]
    mean_pool = oh.T @ h / cnt
    neg = jnp.float32(-jnp.inf)
    max_pool = jnp.stack(
        [jnp.max(jnp.where((batch == b)[:, None], h, neg), axis=0)
         for b in range(B)], axis=0)
    pooled = jnp.concatenate([mean_pool, max_pool], axis=-1)
    mu = pooled @ p['Wmu'] + p['bmu']
    logvar = pooled @ p['Wlv'] + p['blv']
    z = mu + eps * jnp.exp(0.5 * logvar)
    tok = p['tok_emb'][decoder_input_tokens]
    init = (z @ p['Wl2d'] + p['bl2d']).reshape(B, DEC_LAYERS, HID).transpose(1, 0, 2)
    xs = tok.transpose(1, 0, 2)
    for l in range(DEC_LAYERS):
        def step(hprev, x_t, Wih=p['Wih'][l], Whh=p['Whh'][l], bih=p['bih'][l],
                 bhh=p['bhh'][l]):
            gi = x_t @ Wih.T + bih
            gh = hprev @ Whh.T + bhh
            i_r, i_z, i_n = jnp.split(gi, 3, axis=-1)
            h_r, h_z, h_n = jnp.split(gh, 3, axis=-1)
            r = jax.nn.sigmoid(i_r + h_r)
            zg = jax.nn.sigmoid(i_z + h_z)
            n = jnp.tanh(i_n + r * h_n)
            hn = (1.0 - zg) * n + zg * hprev
            return hn, hn
        _, xs = jax.lax.scan(step, init[l], xs)
    out_seq = _ln(xs.transpose(1, 0, 2), p['dn_g'], p['dn_b'])
    token_logits = out_seq @ p['Wtok'] + p['btok']
    size_logits = _gelu(z @ p['Ws1'] + p['bs1']) @ p['Ws2'] + p['bs2']
    return (token_logits, size_logits, mu, logvar, z)
